# baseline (speedup 1.0000x reference)
import functools

import jax
import jax.numpy as jnp
import numpy as np
from jax import lax
from jax.experimental import pallas as pl
from jax.experimental.pallas import tpu as pltpu
from jax.experimental.pallas import tpu_sc as plsc

N_NODE = 10000
E = 320000
D = 128
H = 128

NC = 2
NS = 16
NW = NC * NS

CHUNK = 112
NCH = 180
E_PAD = NCH * NS * CHUNK
NCHT = NCH * NS

_PERM = np.concatenate(
    [np.concatenate([32 * k + 2 * np.arange(16),
                     32 * k + 2 * np.arange(16) + 1]) for k in range(4)])
RPT = 624
RPT_LAST = N_NODE - RPT * (NS - 1)


def _tiled_rowcopy(src, dst, s):
    start = pl.multiple_of(RPT * s, 8)

    @pl.when(s < NS - 1)
    def _():
        pltpu.sync_copy(src.at[pl.ds(start, RPT)], dst.at[pl.ds(start, RPT)])

    @pl.when(s == NS - 1)
    def _():
        pltpu.sync_copy(src.at[pl.ds(RPT * (NS - 1), RPT_LAST)],
                        dst.at[pl.ds(RPT * (NS - 1), RPT_LAST)])


def _agg_side(table, sdr, ewr, agg_out, den_out, z2, z1,
              agg_sh, den_sh, ibufs, gbufs, sbufs, dvec, s, with_den):
    _tiled_rowcopy(z2, agg_sh, s)
    if with_den:
        @pl.when(s < 10)
        def _():
            pltpu.sync_copy(z1.at[pl.ds(1000 * s, 1000)], dvec)
            pltpu.sync_copy(dvec, den_sh.at[pl.ds(1000 * s, 1000)])
    plsc.subcore_barrier()

    base0 = s * (NCH * CHUNK)
    cid0 = s * NCH

    def issue_idx(j, ib):
        sd_b, ew_b, isem = ib
        base = pl.multiple_of(base0 + j * CHUNK, 8)
        pltpu.async_copy(sdr.at[cid0 + j], sd_b, isem)
        pltpu.async_copy(ewr.at[pl.ds(base, CHUNK)], ew_b, isem)

    def wait_idx(j, ib):
        sd_b, ew_b, isem = ib
        base = pl.multiple_of(base0 + j * CHUNK, 8)
        pltpu.make_async_copy(sdr.at[cid0 + j], sd_b, isem).wait()
        pltpu.make_async_copy(ewr.at[pl.ds(base, CHUNK)], ew_b, isem).wait()

    def issue_gather(ib, gb):
        sd_b, ew_b, isem = ib
        rows_bf, gsem = gb
        pltpu.async_copy(table.at[sd_b.at[0]], rows_bf, gsem)

    def wait_scatter(ib, sb):
        sd_b, ew_b, isem = ib
        rows_f, ssem = sb
        pltpu.make_async_copy(rows_f, agg_sh.at[sd_b.at[1]], ssem).wait()
        if with_den:
            pltpu.make_async_copy(ew_b, den_sh.at[sd_b.at[1]], ssem).wait()

    def consume(ib, gb, sb):
        sd_b, ew_b, isem = ib
        rows_bf, gsem = gb
        rows_f, ssem = sb
        pltpu.make_async_copy(table.at[sd_b.at[0]], rows_bf, gsem).wait()

        def scale(e, carry):
            w = plsc.load_gather(ew_b, [jnp.full((16,), e, jnp.int32)])
            for k in range(H // 32):
                v = rows_bf[e, pl.ds(32 * k, 32)]
                a, b = plsc.unpack(v, format=plsc.PackFormat.INTERLEAVED)
                rows_f[e, pl.ds(32 * k, 16)] = a * w
                rows_f[e, pl.ds(32 * k + 16, 16)] = b * w
            return carry
        lax.fori_loop(0, CHUNK, scale, 0, unroll=4)

        pltpu.async_copy(rows_f, agg_sh.at[sd_b.at[1]], ssem, add=True)
        if with_den:
            pltpu.async_copy(ew_b, den_sh.at[sd_b.at[1]], ssem, add=True)

    issue_idx(0, ibufs[0])
    wait_idx(0, ibufs[0])
    issue_gather(ibufs[0], gbufs[0])
    issue_idx(1, ibufs[1])

    def step(j, u):
        ib = ibufs[u]
        gb = gbufs[u % 2]
        sb = sbufs[u % 2]

        @pl.when(j < NCH - 1)
        def _():
            wait_idx(j + 1, ibufs[(u + 1) % 4])
            issue_gather(ibufs[(u + 1) % 4], gbufs[(u + 1) % 2])

        @pl.when(j >= 2)
        def _():
            wait_scatter(ibufs[(u + 2) % 4], sb)
        consume(ib, gb, sb)

        @pl.when(j < NCH - 2)
        def _():
            issue_idx(j + 2, ibufs[(u + 2) % 4])

    def quad(jq, carry):
        for u in range(4):
            step(4 * jq + u, u)
        return carry
    lax.fori_loop(0, NCH // 4, quad, 0)
    wait_scatter(ibufs[(NCH - 2) % 4], sbufs[(NCH - 2) % 2])
    wait_scatter(ibufs[(NCH - 1) % 4], sbufs[(NCH - 1) % 2])

    plsc.subcore_barrier()
    _tiled_rowcopy(agg_sh, agg_out, s)
    if with_den:
        @pl.when(s < 10)
        def _():
            pltpu.sync_copy(den_sh.at[pl.ds(1000 * s, 1000)], dvec)
            pltpu.sync_copy(dvec, den_out.at[pl.ds(1000 * s, 1000)])


def _make_agg_kernel(with_den):
    mesh = plsc.VectorSubcoreMesh(core_axis_name="c", subcore_axis_name="s")

    outs = [jax.ShapeDtypeStruct((N_NODE, H), jnp.float32),
            jax.ShapeDtypeStruct((N_NODE, H), jnp.float32)]
    if with_den:
        outs += [jax.ShapeDtypeStruct((N_NODE,), jnp.float32),
                 jax.ShapeDtypeStruct((N_NODE,), jnp.float32)]

    ibuf_types = [
        pltpu.VMEM((2, CHUNK), jnp.int32),
        pltpu.VMEM((CHUNK,), jnp.float32),
        pltpu.SemaphoreType.DMA,
    ]
    gbuf_types = [
        pltpu.VMEM((CHUNK, H), jnp.bfloat16),
        pltpu.SemaphoreType.DMA,
    ]
    sbuf_types = [
        pltpu.VMEM((CHUNK, H), jnp.float32),
        pltpu.SemaphoreType.DMA,
    ]
    scratch = (
        [pltpu.VMEM_SHARED((N_NODE, H), jnp.float32),
         pltpu.VMEM_SHARED((N_NODE,), jnp.float32)]
        + ibuf_types * 4 + gbuf_types * 2 + sbuf_types * 2
        + [pltpu.VMEM((1000,), jnp.float32)]
    )

    def body(tab0, tab1, sd0, ew0, sd1, ew1, z2, z1, *rest):
        if with_den:
            agg_u, agg_a, den_u, den_a = rest[:4]
            scr = rest[4:]
        else:
            agg_u, agg_a = rest[:2]
            den_u = den_a = None
            scr = rest[2:]
        agg_sh, den_sh = scr[0], scr[1]
        ibufs = tuple(tuple(scr[2 + 3 * i:5 + 3 * i]) for i in range(4))
        gbufs = tuple(tuple(scr[14 + 2 * i:16 + 2 * i]) for i in range(2))
        sbufs = tuple(tuple(scr[18 + 2 * i:20 + 2 * i]) for i in range(2))
        dvec = scr[22]
        c = lax.axis_index("c")
        s = lax.axis_index("s")

        @pl.when(c == 0)
        def _():
            _agg_side(tab0, sd0, ew0, agg_u, den_u, z2, z1,
                      agg_sh, den_sh, ibufs, gbufs, sbufs, dvec, s, with_den)

        @pl.when(c == 1)
        def _():
            _agg_side(tab1, sd1, ew1, agg_a, den_a, z2, z1,
                      agg_sh, den_sh, ibufs, gbufs, sbufs, dvec, s, with_den)

    return pl.kernel(body, out_type=tuple(outs), mesh=mesh,
                     scratch_types=scratch,
                     compiler_params=pltpu.CompilerParams(
                         needs_layout_passes=False,
                         use_tc_tiling_on_sc=False),
                     name="sc_edge_agg_den" if with_den else "sc_edge_agg")


_agg_kernel_l0 = _make_agg_kernel(True)
_agg_kernel_l1 = _make_agg_kernel(False)


EPW = E // NW


def _score_body(su_hbm, sa_hbm, src_hbm, dst_hbm, out_hbm,
                su_v, sa_v, src_v, dst_v, out_v):
    c = lax.axis_index("c")
    s = lax.axis_index("s")
    w = s * NC + c
    pltpu.sync_copy(su_hbm, su_v)
    pltpu.sync_copy(sa_hbm, sa_v)
    base = w * EPW
    pltpu.sync_copy(src_hbm.at[pl.ds(base, EPW)], src_v)
    pltpu.sync_copy(dst_hbm.at[pl.ds(base, EPW)], dst_v)

    def it(i, carry):
        sl = pl.ds(i * 16, 16)
        vu = plsc.load_gather(su_v, [src_v[sl]])
        va = plsc.load_gather(sa_v, [dst_v[sl]])
        out_v[sl] = vu + va
        return carry
    lax.fori_loop(0, EPW // 16, it, 0)
    pltpu.sync_copy(out_v, out_hbm.at[pl.ds(base, EPW)])


_score_kernel = pl.kernel(
    _score_body,
    out_type=jax.ShapeDtypeStruct((E,), jnp.float32),
    mesh=plsc.VectorSubcoreMesh(core_axis_name="c", subcore_axis_name="s"),
    scratch_types=[
        pltpu.VMEM((N_NODE,), jnp.float32),
        pltpu.VMEM((N_NODE,), jnp.float32),
        pltpu.VMEM((EPW,), jnp.int32),
        pltpu.VMEM((EPW,), jnp.int32),
        pltpu.VMEM((EPW,), jnp.float32),
    ],
    compiler_params=pltpu.CompilerParams(needs_layout_passes=False),
    name="sc_edge_score",
)


BR = 1000
GRID = N_NODE // BR


def _l0_body(xu, xa, aggu, agga, denu, dena,
             Wsu, Wnu, bu, Wsa, Wna, ba,
             hu_ref, ha_ref, hu_bf_ref, ha_bf_ref):
    du = jnp.maximum(denu[...], 1e-6)
    da = jnp.maximum(dena[...], 1e-6)
    au = aggu[...] / du
    aa = agga[...] / da
    hu = jnp.maximum(xu[...] @ Wsu[...] + au @ Wnu[...] + bu[...], 0.0)
    ha = jnp.maximum(xa[...] @ Wsa[...] + aa @ Wna[...] + ba[...], 0.0)
    hu_ref[...] = hu
    ha_ref[...] = ha
    hu_bf_ref[...] = hu.astype(jnp.bfloat16)
    ha_bf_ref[...] = ha.astype(jnp.bfloat16)


def _l1_body(hu, ha, aggu, agga, denu, dena,
             Wsu, Wnu, bu, Wsa, Wna, ba,
             luW, lub, liW, lib, oWu, oWi, ob, su_ref, sa_ref):
    du = jnp.maximum(denu[...], 1e-6)
    da = jnp.maximum(dena[...], 1e-6)
    zu = hu[...] @ Wsu[...] + (aggu[...] / du) @ Wnu[...] + bu[...]
    za = ha[...] @ Wsa[...] + (agga[...] / da) @ Wna[...] + ba[...]
    uf = zu @ luW[...] + lub[...]
    itf = za @ liW[...] + lib[...]
    su_ref[...] = uf @ oWu[...] + ob[...]
    sa_ref[...] = itf @ oWi[...]


def _row_spec(ncol):
    return pl.BlockSpec((BR, ncol), lambda i: (i, 0))


def _full_spec(shape):
    nd = len(shape)
    return pl.BlockSpec(shape, lambda i: (0,) * nd)


_l0_call = pl.pallas_call(
    _l0_body,
    grid=(GRID,),
    in_specs=[_row_spec(H), _row_spec(H), _row_spec(H), _row_spec(H),
              _row_spec(1), _row_spec(1),
              _full_spec((H, H)), _full_spec((H, H)), _full_spec((1, H)),
              _full_spec((H, H)), _full_spec((H, H)), _full_spec((1, H))],
    out_specs=[_row_spec(H), _row_spec(H), _row_spec(H), _row_spec(H)],
    out_shape=[jax.ShapeDtypeStruct((N_NODE, H), jnp.float32),
               jax.ShapeDtypeStruct((N_NODE, H), jnp.float32),
               jax.ShapeDtypeStruct((N_NODE, H), jnp.bfloat16),
               jax.ShapeDtypeStruct((N_NODE, H), jnp.bfloat16)],
)

_l1_call = pl.pallas_call(
    _l1_body,
    grid=(GRID,),
    in_specs=[_row_spec(H), _row_spec(H), _row_spec(H), _row_spec(H),
              _row_spec(1), _row_spec(1),
              _full_spec((H, H)), _full_spec((H, H)), _full_spec((1, H)),
              _full_spec((H, H)), _full_spec((H, H)), _full_spec((1, H)),
              _full_spec((H, H // 2)), _full_spec((1, H // 2)),
              _full_spec((H, H // 2)), _full_spec((1, H // 2)),
              _full_spec((H // 2, 1)), _full_spec((H // 2, 1)),
              _full_spec((1, 1))],
    out_specs=[_row_spec(1), _row_spec(1)],
    out_shape=[jax.ShapeDtypeStruct((N_NODE, 1), jnp.float32),
               jax.ShapeDtypeStruct((N_NODE, 1), jnp.float32)],
)


def kernel(x_user, x_artwork, edge_index_rates, edge_index_rev,
           edge_weight_rates, edge_weight_rev,
           W_self_user_l0, W_neigh_user_l0, b_user_l0,
           W_self_art_l0, W_neigh_art_l0, b_art_l0,
           W_self_user_l1, W_neigh_user_l1, b_user_l1,
           W_self_art_l1, W_neigh_art_l1, b_art_l1,
           lin_user_W, lin_user_b, lin_item_W, lin_item_b, out_W, out_b):
    src_rev = edge_index_rev[0]
    dst_rev = edge_index_rev[1]
    src_rts = edge_index_rates[0]
    dst_rts = edge_index_rates[1]
    z2 = jnp.zeros((N_NODE, H), jnp.float32)
    z1 = jnp.zeros((N_NODE,), jnp.float32)

    pad_i = [(0, E_PAD - E)]

    def pack_sd(src, dst):
        return jnp.stack([jnp.pad(src, pad_i).reshape(NCHT, CHUNK),
                          jnp.pad(dst, pad_i).reshape(NCHT, CHUNK)], axis=1)

    sd_rev = pack_sd(src_rev, dst_rev)
    ew_rev_p = jnp.pad(edge_weight_rev, pad_i)
    sd_rts = pack_sd(src_rts, dst_rts)
    ew_rts_p = jnp.pad(edge_weight_rates, pad_i)

    agg_u0, agg_a0, den_u, den_a = _agg_kernel_l0(
        x_artwork.astype(jnp.bfloat16), x_user.astype(jnp.bfloat16),
        sd_rev, ew_rev_p, sd_rts, ew_rts_p, z2, z1)

    den_u2 = den_u.reshape(N_NODE, 1)
    den_a2 = den_a.reshape(N_NODE, 1)
    bu0 = b_user_l0.reshape(1, H)
    ba0 = b_art_l0.reshape(1, H)
    hu, ha, hu_bf, ha_bf = _l0_call(
        x_user, x_artwork, agg_u0, agg_a0, den_u2, den_a2,
        W_self_user_l0, W_neigh_user_l0[_PERM], bu0,
        W_self_art_l0, W_neigh_art_l0[_PERM], ba0)

    agg_u1, agg_a1 = _agg_kernel_l1(
        ha_bf, hu_bf, sd_rev, ew_rev_p, sd_rts, ew_rts_p, z2, z1)

    su, sa = _l1_call(
        hu, ha, agg_u1, agg_a1, den_u2, den_a2,
        W_self_user_l1, W_neigh_user_l1[_PERM], b_user_l1.reshape(1, H),
        W_self_art_l1, W_neigh_art_l1[_PERM], b_art_l1.reshape(1, H),
        lin_user_W, lin_user_b.reshape(1, H // 2),
        lin_item_W, lin_item_b.reshape(1, H // 2),
        out_W[:H // 2], out_W[H // 2:], out_b.reshape(1, 1))

    out = _score_kernel(su.reshape(N_NODE), sa.reshape(N_NODE),
                        src_rts, dst_rts)
    return out.reshape(E, 1)


# speedup vs baseline: 1.4864x; 1.4864x over previous
import functools

import jax
import jax.numpy as jnp
import numpy as np
from jax import lax
from jax.experimental import pallas as pl
from jax.experimental.pallas import tpu as pltpu
from jax.experimental.pallas import tpu_sc as plsc

N_NODE = 10000
E = 320000
D = 128
H = 128

NC = 2
NS = 16
NW = NC * NS

CHUNK = 112
NCH = 180
E_PAD = NCH * NS * CHUNK
NCHT = NCH * NS
RPT = 624
RPT_LAST = N_NODE - RPT * (NS - 1)


def _tiled_rowcopy(src, dst, s):
    start = pl.multiple_of(RPT * s, 8)

    @pl.when(s < NS - 1)
    def _():
        pltpu.sync_copy(src.at[pl.ds(start, RPT)], dst.at[pl.ds(start, RPT)])

    @pl.when(s == NS - 1)
    def _():
        pltpu.sync_copy(src.at[pl.ds(RPT * (NS - 1), RPT_LAST)],
                        dst.at[pl.ds(RPT * (NS - 1), RPT_LAST)])


def _agg_side(table, sdr, ewr, agg_out, den_out, z2, z1,
              agg_sh, den_sh, ibufs, dvec, s, with_den):
    _tiled_rowcopy(z2, agg_sh, s)
    if with_den:
        @pl.when(s < 10)
        def _():
            pltpu.sync_copy(z1.at[pl.ds(1000 * s, 1000)], dvec)
            pltpu.sync_copy(dvec, den_sh.at[pl.ds(1000 * s, 1000)])
    plsc.subcore_barrier()

    base0 = s * (NCH * CHUNK)
    cid0 = s * NCH

    def issue_idx(j, buf):
        sd_b, ew_b, rows_b, isem, gsem, ssem = buf
        base = pl.multiple_of(base0 + j * CHUNK, 8)
        pltpu.async_copy(sdr.at[cid0 + j], sd_b, isem)
        pltpu.async_copy(ewr.at[pl.ds(base, CHUNK)], ew_b, isem)

    def wait_idx(j, buf):
        sd_b, ew_b, rows_b, isem, gsem, ssem = buf
        base = pl.multiple_of(base0 + j * CHUNK, 8)
        pltpu.make_async_copy(sdr.at[cid0 + j], sd_b, isem).wait()
        pltpu.make_async_copy(ewr.at[pl.ds(base, CHUNK)], ew_b, isem).wait()

    def issue_gather(buf):
        sd_b, ew_b, rows_b, isem, gsem, ssem = buf
        pltpu.async_copy(table.at[sd_b.at[0]], rows_b, gsem)

    def wait_scatter(buf):
        sd_b, ew_b, rows_b, isem, gsem, ssem = buf
        pltpu.make_async_copy(rows_b, agg_sh.at[sd_b.at[1]], ssem).wait()
        if with_den:
            pltpu.make_async_copy(ew_b, den_sh.at[sd_b.at[1]], ssem).wait()

    def consume(buf):
        sd_b, ew_b, rows_b, isem, gsem, ssem = buf
        pltpu.make_async_copy(table.at[sd_b.at[0]], rows_b, gsem).wait()

        def scale(e, carry):
            w = plsc.load_gather(ew_b, [jnp.full((16,), e, jnp.int32)])
            for k in range(H // 16):
                sl = pl.ds(16 * k, 16)
                rows_b[e, sl] = rows_b[e, sl] * w
            return carry
        lax.fori_loop(0, CHUNK, scale, 0, unroll=4)

        pltpu.async_copy(rows_b, agg_sh.at[sd_b.at[1]], ssem, add=True)
        if with_den:
            pltpu.async_copy(ew_b, den_sh.at[sd_b.at[1]], ssem, add=True)

    bufs = ibufs
    issue_idx(0, bufs[0])
    wait_idx(0, bufs[0])
    issue_gather(bufs[0])
    issue_idx(1, bufs[1])

    def step(j, u):
        bj = bufs[u]
        bn = bufs[(u + 1) % 3]
        bp = bufs[(u + 2) % 3]

        @pl.when(j < NCH - 1)
        def _():
            wait_idx(j + 1, bn)
            issue_gather(bn)
        consume(bj)

        @pl.when(j > 0)
        def _():
            wait_scatter(bp)

        @pl.when(j < NCH - 2)
        def _():
            issue_idx(j + 2, bp)

    def triple(jt, carry):
        for u in range(3):
            step(3 * jt + u, u)
        return carry
    lax.fori_loop(0, NCH // 3, triple, 0)
    wait_scatter(bufs[(NCH - 1) % 3])

    plsc.subcore_barrier()
    _tiled_rowcopy(agg_sh, agg_out, s)
    if with_den:
        @pl.when(s < 10)
        def _():
            pltpu.sync_copy(den_sh.at[pl.ds(1000 * s, 1000)], dvec)
            pltpu.sync_copy(dvec, den_out.at[pl.ds(1000 * s, 1000)])


def _make_agg_kernel(with_den):
    mesh = plsc.VectorSubcoreMesh(core_axis_name="c", subcore_axis_name="s")

    outs = [jax.ShapeDtypeStruct((N_NODE, H), jnp.float32),
            jax.ShapeDtypeStruct((N_NODE, H), jnp.float32)]
    if with_den:
        outs += [jax.ShapeDtypeStruct((N_NODE,), jnp.float32),
                 jax.ShapeDtypeStruct((N_NODE,), jnp.float32)]

    buf_types = [
        pltpu.VMEM((2, CHUNK), jnp.int32),
        pltpu.VMEM((CHUNK,), jnp.float32),
        pltpu.VMEM((CHUNK, H), jnp.float32),
        pltpu.SemaphoreType.DMA,
        pltpu.SemaphoreType.DMA,
        pltpu.SemaphoreType.DMA,
    ]
    scratch = (
        [pltpu.VMEM_SHARED((N_NODE, H), jnp.float32),
         pltpu.VMEM_SHARED((N_NODE,), jnp.float32)]
        + buf_types * 3
        + [pltpu.VMEM((1000,), jnp.float32)]
    )

    def body(tab0, tab1, sd0, ew0, sd1, ew1, z2, z1, *rest):
        if with_den:
            agg_u, agg_a, den_u, den_a = rest[:4]
            scr = rest[4:]
        else:
            agg_u, agg_a = rest[:2]
            den_u = den_a = None
            scr = rest[2:]
        agg_sh, den_sh = scr[0], scr[1]
        ibufs = tuple(tuple(scr[2 + 6 * i:8 + 6 * i]) for i in range(3))
        dvec = scr[20]
        c = lax.axis_index("c")
        s = lax.axis_index("s")

        @pl.when(c == 0)
        def _():
            _agg_side(tab0, sd0, ew0, agg_u, den_u, z2, z1,
                      agg_sh, den_sh, ibufs, dvec, s, with_den)

        @pl.when(c == 1)
        def _():
            _agg_side(tab1, sd1, ew1, agg_a, den_a, z2, z1,
                      agg_sh, den_sh, ibufs, dvec, s, with_den)

    return pl.kernel(body, out_type=tuple(outs), mesh=mesh,
                     scratch_types=scratch,
                     compiler_params=pltpu.CompilerParams(
                         needs_layout_passes=False),
                     name="sc_edge_agg_den" if with_den else "sc_edge_agg")


_agg_kernel_l0 = _make_agg_kernel(True)
_agg_kernel_l1 = _make_agg_kernel(False)


EPW = E // NW


def _score_body(su_hbm, sa_hbm, src_hbm, dst_hbm, out_hbm,
                su_v, sa_v, src_v, dst_v, out_v):
    c = lax.axis_index("c")
    s = lax.axis_index("s")
    w = s * NC + c
    pltpu.sync_copy(su_hbm, su_v)
    pltpu.sync_copy(sa_hbm, sa_v)
    base = w * EPW
    pltpu.sync_copy(src_hbm.at[pl.ds(base, EPW)], src_v)
    pltpu.sync_copy(dst_hbm.at[pl.ds(base, EPW)], dst_v)

    def it(i, carry):
        sl = pl.ds(i * 16, 16)
        vu = plsc.load_gather(su_v, [src_v[sl]])
        va = plsc.load_gather(sa_v, [dst_v[sl]])
        out_v[sl] = vu + va
        return carry
    lax.fori_loop(0, EPW // 16, it, 0)
    pltpu.sync_copy(out_v, out_hbm.at[pl.ds(base, EPW)])


_score_kernel = pl.kernel(
    _score_body,
    out_type=jax.ShapeDtypeStruct((E,), jnp.float32),
    mesh=plsc.VectorSubcoreMesh(core_axis_name="c", subcore_axis_name="s"),
    scratch_types=[
        pltpu.VMEM((N_NODE,), jnp.float32),
        pltpu.VMEM((N_NODE,), jnp.float32),
        pltpu.VMEM((EPW,), jnp.int32),
        pltpu.VMEM((EPW,), jnp.int32),
        pltpu.VMEM((EPW,), jnp.float32),
    ],
    compiler_params=pltpu.CompilerParams(needs_layout_passes=False),
    name="sc_edge_score",
)


BR = 1000
GRID = N_NODE // BR


def _l0_body(xu, xa, aggu, agga, denu, dena,
             Wsu, Wnu, bu, Wsa, Wna, ba, hu_ref, ha_ref):
    du = jnp.maximum(denu[...], 1e-6)
    da = jnp.maximum(dena[...], 1e-6)
    au = aggu[...] / du
    aa = agga[...] / da
    hu_ref[...] = jnp.maximum(xu[...] @ Wsu[...] + au @ Wnu[...] + bu[...], 0.0)
    ha_ref[...] = jnp.maximum(xa[...] @ Wsa[...] + aa @ Wna[...] + ba[...], 0.0)


def _l1_body(hu, ha, aggu, agga, denu, dena,
             Wsu, Wnu, bu, Wsa, Wna, ba,
             luW, lub, liW, lib, oWu, oWi, ob, su_ref, sa_ref):
    du = jnp.maximum(denu[...], 1e-6)
    da = jnp.maximum(dena[...], 1e-6)
    zu = hu[...] @ Wsu[...] + (aggu[...] / du) @ Wnu[...] + bu[...]
    za = ha[...] @ Wsa[...] + (agga[...] / da) @ Wna[...] + ba[...]
    uf = zu @ luW[...] + lub[...]
    itf = za @ liW[...] + lib[...]
    su_ref[...] = uf @ oWu[...] + ob[...]
    sa_ref[...] = itf @ oWi[...]


def _row_spec(ncol):
    return pl.BlockSpec((BR, ncol), lambda i: (i, 0))


def _full_spec(shape):
    nd = len(shape)
    return pl.BlockSpec(shape, lambda i: (0,) * nd)


_l0_call = pl.pallas_call(
    _l0_body,
    grid=(GRID,),
    in_specs=[_row_spec(H), _row_spec(H), _row_spec(H), _row_spec(H),
              _row_spec(1), _row_spec(1),
              _full_spec((H, H)), _full_spec((H, H)), _full_spec((1, H)),
              _full_spec((H, H)), _full_spec((H, H)), _full_spec((1, H))],
    out_specs=[_row_spec(H), _row_spec(H)],
    out_shape=[jax.ShapeDtypeStruct((N_NODE, H), jnp.float32),
               jax.ShapeDtypeStruct((N_NODE, H), jnp.float32)],
)

_l1_call = pl.pallas_call(
    _l1_body,
    grid=(GRID,),
    in_specs=[_row_spec(H), _row_spec(H), _row_spec(H), _row_spec(H),
              _row_spec(1), _row_spec(1),
              _full_spec((H, H)), _full_spec((H, H)), _full_spec((1, H)),
              _full_spec((H, H)), _full_spec((H, H)), _full_spec((1, H)),
              _full_spec((H, H // 2)), _full_spec((1, H // 2)),
              _full_spec((H, H // 2)), _full_spec((1, H // 2)),
              _full_spec((H // 2, 1)), _full_spec((H // 2, 1)),
              _full_spec((1, 1))],
    out_specs=[_row_spec(1), _row_spec(1)],
    out_shape=[jax.ShapeDtypeStruct((N_NODE, 1), jnp.float32),
               jax.ShapeDtypeStruct((N_NODE, 1), jnp.float32)],
)


def kernel(x_user, x_artwork, edge_index_rates, edge_index_rev,
           edge_weight_rates, edge_weight_rev,
           W_self_user_l0, W_neigh_user_l0, b_user_l0,
           W_self_art_l0, W_neigh_art_l0, b_art_l0,
           W_self_user_l1, W_neigh_user_l1, b_user_l1,
           W_self_art_l1, W_neigh_art_l1, b_art_l1,
           lin_user_W, lin_user_b, lin_item_W, lin_item_b, out_W, out_b):
    src_rev = edge_index_rev[0]
    dst_rev = edge_index_rev[1]
    src_rts = edge_index_rates[0]
    dst_rts = edge_index_rates[1]
    z2 = jnp.zeros((N_NODE, H), jnp.float32)
    z1 = jnp.zeros((N_NODE,), jnp.float32)

    pad_i = [(0, E_PAD - E)]

    def pack_sd(src, dst):
        return jnp.stack([jnp.pad(src, pad_i).reshape(NCHT, CHUNK),
                          jnp.pad(dst, pad_i).reshape(NCHT, CHUNK)], axis=1)

    sd_rev = pack_sd(src_rev, dst_rev)
    ew_rev_p = jnp.pad(edge_weight_rev, pad_i)
    sd_rts = pack_sd(src_rts, dst_rts)
    ew_rts_p = jnp.pad(edge_weight_rates, pad_i)

    agg_u0, agg_a0, den_u, den_a = _agg_kernel_l0(
        x_artwork, x_user, sd_rev, ew_rev_p, sd_rts, ew_rts_p, z2, z1)

    den_u2 = den_u.reshape(N_NODE, 1)
    den_a2 = den_a.reshape(N_NODE, 1)
    bu0 = b_user_l0.reshape(1, H)
    ba0 = b_art_l0.reshape(1, H)
    hu, ha = _l0_call(
        x_user, x_artwork, agg_u0, agg_a0, den_u2, den_a2,
        W_self_user_l0, W_neigh_user_l0, bu0,
        W_self_art_l0, W_neigh_art_l0, ba0)

    agg_u1, agg_a1 = _agg_kernel_l1(
        ha, hu, sd_rev, ew_rev_p, sd_rts, ew_rts_p, z2, z1)

    su, sa = _l1_call(
        hu, ha, agg_u1, agg_a1, den_u2, den_a2,
        W_self_user_l1, W_neigh_user_l1, b_user_l1.reshape(1, H),
        W_self_art_l1, W_neigh_art_l1, b_art_l1.reshape(1, H),
        lin_user_W, lin_user_b.reshape(1, H // 2),
        lin_item_W, lin_item_b.reshape(1, H // 2),
        out_W[:H // 2], out_W[H // 2:], out_b.reshape(1, 1))

    out = _score_kernel(su.reshape(N_NODE), sa.reshape(N_NODE),
                        src_rts, dst_rts)
    return out.reshape(E, 1)


# speedup vs baseline: 1.5092x; 1.0154x over previous
import functools

import jax
import jax.numpy as jnp
import numpy as np
from jax import lax
from jax.experimental import pallas as pl
from jax.experimental.pallas import tpu as pltpu
from jax.experimental.pallas import tpu_sc as plsc

N_NODE = 10000
E = 320000
D = 128
H = 128

NC = 2
NS = 16
NW = NC * NS

CHUNK = 120
NCH = 168
E_PAD = NCH * NS * CHUNK
NCHT = NCH * NS
RPT = 624
RPT_LAST = N_NODE - RPT * (NS - 1)


def _tiled_rowcopy(src, dst, s):
    start = pl.multiple_of(RPT * s, 8)

    @pl.when(s < NS - 1)
    def _():
        pltpu.sync_copy(src.at[pl.ds(start, RPT)], dst.at[pl.ds(start, RPT)])

    @pl.when(s == NS - 1)
    def _():
        pltpu.sync_copy(src.at[pl.ds(RPT * (NS - 1), RPT_LAST)],
                        dst.at[pl.ds(RPT * (NS - 1), RPT_LAST)])


def _agg_side(table, sdr, ewr, agg_out, den_out, z2, z1,
              agg_sh, den_sh, ibufs, dvec, s, with_den):
    _tiled_rowcopy(z2, agg_sh, s)
    if with_den:
        @pl.when(s < 10)
        def _():
            pltpu.sync_copy(z1.at[pl.ds(1000 * s, 1000)], dvec)
            pltpu.sync_copy(dvec, den_sh.at[pl.ds(1000 * s, 1000)])
    plsc.subcore_barrier()

    base0 = s * (NCH * CHUNK)
    cid0 = s * NCH

    def issue_idx(j, buf):
        sd_b, ew_b, rows_b, isem, gsem, ssem = buf
        base = pl.multiple_of(base0 + j * CHUNK, 8)
        pltpu.async_copy(sdr.at[cid0 + j], sd_b, isem)
        pltpu.async_copy(ewr.at[pl.ds(base, CHUNK)], ew_b, isem)

    def wait_idx(j, buf):
        sd_b, ew_b, rows_b, isem, gsem, ssem = buf
        base = pl.multiple_of(base0 + j * CHUNK, 8)
        pltpu.make_async_copy(sdr.at[cid0 + j], sd_b, isem).wait()
        pltpu.make_async_copy(ewr.at[pl.ds(base, CHUNK)], ew_b, isem).wait()

    def issue_gather(buf):
        sd_b, ew_b, rows_b, isem, gsem, ssem = buf
        pltpu.async_copy(table.at[sd_b.at[0]], rows_b, gsem)

    def wait_scatter(buf):
        sd_b, ew_b, rows_b, isem, gsem, ssem = buf
        pltpu.make_async_copy(rows_b, agg_sh.at[sd_b.at[1]], ssem).wait()
        if with_den:
            pltpu.make_async_copy(ew_b, den_sh.at[sd_b.at[1]], ssem).wait()

    def consume(buf):
        sd_b, ew_b, rows_b, isem, gsem, ssem = buf
        pltpu.make_async_copy(table.at[sd_b.at[0]], rows_b, gsem).wait()

        def scale(e, carry):
            w = plsc.load_gather(ew_b, [jnp.full((16,), e, jnp.int32)])
            for k in range(H // 16):
                sl = pl.ds(16 * k, 16)
                rows_b[e, sl] = rows_b[e, sl] * w
            return carry
        lax.fori_loop(0, CHUNK, scale, 0, unroll=8)

        pltpu.async_copy(rows_b, agg_sh.at[sd_b.at[1]], ssem, add=True)
        if with_den:
            pltpu.async_copy(ew_b, den_sh.at[sd_b.at[1]], ssem, add=True)

    bufs = ibufs
    issue_idx(0, bufs[0])
    wait_idx(0, bufs[0])
    issue_gather(bufs[0])
    issue_idx(1, bufs[1])

    def step(j, u):
        bj = bufs[u]
        bn = bufs[(u + 1) % 3]
        bp = bufs[(u + 2) % 3]

        @pl.when(j < NCH - 1)
        def _():
            wait_idx(j + 1, bn)
            issue_gather(bn)
        consume(bj)

        @pl.when(j > 0)
        def _():
            wait_scatter(bp)

        @pl.when(j < NCH - 2)
        def _():
            issue_idx(j + 2, bp)

    def triple(jt, carry):
        for u in range(3):
            step(3 * jt + u, u)
        return carry
    lax.fori_loop(0, NCH // 3, triple, 0)
    wait_scatter(bufs[(NCH - 1) % 3])

    plsc.subcore_barrier()
    _tiled_rowcopy(agg_sh, agg_out, s)
    if with_den:
        @pl.when(s < 10)
        def _():
            pltpu.sync_copy(den_sh.at[pl.ds(1000 * s, 1000)], dvec)
            pltpu.sync_copy(dvec, den_out.at[pl.ds(1000 * s, 1000)])


def _make_agg_kernel(with_den):
    mesh = plsc.VectorSubcoreMesh(core_axis_name="c", subcore_axis_name="s")

    outs = [jax.ShapeDtypeStruct((N_NODE, H), jnp.float32),
            jax.ShapeDtypeStruct((N_NODE, H), jnp.float32)]
    if with_den:
        outs += [jax.ShapeDtypeStruct((N_NODE,), jnp.float32),
                 jax.ShapeDtypeStruct((N_NODE,), jnp.float32)]

    buf_types = [
        pltpu.VMEM((2, CHUNK), jnp.int32),
        pltpu.VMEM((CHUNK,), jnp.float32),
        pltpu.VMEM((CHUNK, H), jnp.float32),
        pltpu.SemaphoreType.DMA,
        pltpu.SemaphoreType.DMA,
        pltpu.SemaphoreType.DMA,
    ]
    scratch = (
        [pltpu.VMEM_SHARED((N_NODE, H), jnp.float32),
         pltpu.VMEM_SHARED((N_NODE,), jnp.float32)]
        + buf_types * 3
        + [pltpu.VMEM((1000,), jnp.float32)]
    )

    def body(tab0, tab1, sd0, ew0, sd1, ew1, z2, z1, *rest):
        if with_den:
            agg_u, agg_a, den_u, den_a = rest[:4]
            scr = rest[4:]
        else:
            agg_u, agg_a = rest[:2]
            den_u = den_a = None
            scr = rest[2:]
        agg_sh, den_sh = scr[0], scr[1]
        ibufs = tuple(tuple(scr[2 + 6 * i:8 + 6 * i]) for i in range(3))
        dvec = scr[20]
        c = lax.axis_index("c")
        s = lax.axis_index("s")

        @pl.when(c == 0)
        def _():
            _agg_side(tab0, sd0, ew0, agg_u, den_u, z2, z1,
                      agg_sh, den_sh, ibufs, dvec, s, with_den)

        @pl.when(c == 1)
        def _():
            _agg_side(tab1, sd1, ew1, agg_a, den_a, z2, z1,
                      agg_sh, den_sh, ibufs, dvec, s, with_den)

    return pl.kernel(body, out_type=tuple(outs), mesh=mesh,
                     scratch_types=scratch,
                     compiler_params=pltpu.CompilerParams(
                         needs_layout_passes=False),
                     name="sc_edge_agg_den" if with_den else "sc_edge_agg")


_agg_kernel_l0 = _make_agg_kernel(True)
_agg_kernel_l1 = _make_agg_kernel(False)


EPW = E // NW


def _score_body(su_hbm, sa_hbm, src_hbm, dst_hbm, out_hbm,
                su_v, sa_v, src_v, dst_v, out_v):
    c = lax.axis_index("c")
    s = lax.axis_index("s")
    w = s * NC + c
    pltpu.sync_copy(su_hbm, su_v)
    pltpu.sync_copy(sa_hbm, sa_v)
    base = w * EPW
    pltpu.sync_copy(src_hbm.at[pl.ds(base, EPW)], src_v)
    pltpu.sync_copy(dst_hbm.at[pl.ds(base, EPW)], dst_v)

    def it(i, carry):
        sl = pl.ds(i * 16, 16)
        vu = plsc.load_gather(su_v, [src_v[sl]])
        va = plsc.load_gather(sa_v, [dst_v[sl]])
        out_v[sl] = vu + va
        return carry
    lax.fori_loop(0, EPW // 16, it, 0)
    pltpu.sync_copy(out_v, out_hbm.at[pl.ds(base, EPW)])


_score_kernel = pl.kernel(
    _score_body,
    out_type=jax.ShapeDtypeStruct((E,), jnp.float32),
    mesh=plsc.VectorSubcoreMesh(core_axis_name="c", subcore_axis_name="s"),
    scratch_types=[
        pltpu.VMEM((N_NODE,), jnp.float32),
        pltpu.VMEM((N_NODE,), jnp.float32),
        pltpu.VMEM((EPW,), jnp.int32),
        pltpu.VMEM((EPW,), jnp.int32),
        pltpu.VMEM((EPW,), jnp.float32),
    ],
    compiler_params=pltpu.CompilerParams(needs_layout_passes=False),
    name="sc_edge_score",
)


BR = 1000
GRID = N_NODE // BR


def _l0_body(xu, xa, aggu, agga, denu, dena,
             Wsu, Wnu, bu, Wsa, Wna, ba, hu_ref, ha_ref):
    du = jnp.maximum(denu[...], 1e-6)
    da = jnp.maximum(dena[...], 1e-6)
    au = aggu[...] / du
    aa = agga[...] / da
    hu_ref[...] = jnp.maximum(xu[...] @ Wsu[...] + au @ Wnu[...] + bu[...], 0.0)
    ha_ref[...] = jnp.maximum(xa[...] @ Wsa[...] + aa @ Wna[...] + ba[...], 0.0)


def _l1_body(hu, ha, aggu, agga, denu, dena,
             Wsu, Wnu, bu, Wsa, Wna, ba,
             luW, lub, liW, lib, oWu, oWi, ob, su_ref, sa_ref):
    du = jnp.maximum(denu[...], 1e-6)
    da = jnp.maximum(dena[...], 1e-6)
    zu = hu[...] @ Wsu[...] + (aggu[...] / du) @ Wnu[...] + bu[...]
    za = ha[...] @ Wsa[...] + (agga[...] / da) @ Wna[...] + ba[...]
    uf = zu @ luW[...] + lub[...]
    itf = za @ liW[...] + lib[...]
    su_ref[...] = uf @ oWu[...] + ob[...]
    sa_ref[...] = itf @ oWi[...]


def _row_spec(ncol):
    return pl.BlockSpec((BR, ncol), lambda i: (i, 0))


def _full_spec(shape):
    nd = len(shape)
    return pl.BlockSpec(shape, lambda i: (0,) * nd)


_l0_call = pl.pallas_call(
    _l0_body,
    grid=(GRID,),
    in_specs=[_row_spec(H), _row_spec(H), _row_spec(H), _row_spec(H),
              _row_spec(1), _row_spec(1),
              _full_spec((H, H)), _full_spec((H, H)), _full_spec((1, H)),
              _full_spec((H, H)), _full_spec((H, H)), _full_spec((1, H))],
    out_specs=[_row_spec(H), _row_spec(H)],
    out_shape=[jax.ShapeDtypeStruct((N_NODE, H), jnp.float32),
               jax.ShapeDtypeStruct((N_NODE, H), jnp.float32)],
)

_l1_call = pl.pallas_call(
    _l1_body,
    grid=(GRID,),
    in_specs=[_row_spec(H), _row_spec(H), _row_spec(H), _row_spec(H),
              _row_spec(1), _row_spec(1),
              _full_spec((H, H)), _full_spec((H, H)), _full_spec((1, H)),
              _full_spec((H, H)), _full_spec((H, H)), _full_spec((1, H)),
              _full_spec((H, H // 2)), _full_spec((1, H // 2)),
              _full_spec((H, H // 2)), _full_spec((1, H // 2)),
              _full_spec((H // 2, 1)), _full_spec((H // 2, 1)),
              _full_spec((1, 1))],
    out_specs=[_row_spec(1), _row_spec(1)],
    out_shape=[jax.ShapeDtypeStruct((N_NODE, 1), jnp.float32),
               jax.ShapeDtypeStruct((N_NODE, 1), jnp.float32)],
)


def kernel(x_user, x_artwork, edge_index_rates, edge_index_rev,
           edge_weight_rates, edge_weight_rev,
           W_self_user_l0, W_neigh_user_l0, b_user_l0,
           W_self_art_l0, W_neigh_art_l0, b_art_l0,
           W_self_user_l1, W_neigh_user_l1, b_user_l1,
           W_self_art_l1, W_neigh_art_l1, b_art_l1,
           lin_user_W, lin_user_b, lin_item_W, lin_item_b, out_W, out_b):
    src_rev = edge_index_rev[0]
    dst_rev = edge_index_rev[1]
    src_rts = edge_index_rates[0]
    dst_rts = edge_index_rates[1]
    z2 = jnp.zeros((N_NODE, H), jnp.float32)
    z1 = jnp.zeros((N_NODE,), jnp.float32)

    pad_i = [(0, E_PAD - E)]

    def pack_sd(src, dst):
        return jnp.stack([jnp.pad(src, pad_i).reshape(NCHT, CHUNK),
                          jnp.pad(dst, pad_i).reshape(NCHT, CHUNK)], axis=1)

    sd_rev = pack_sd(src_rev, dst_rev)
    ew_rev_p = jnp.pad(edge_weight_rev, pad_i)
    sd_rts = pack_sd(src_rts, dst_rts)
    ew_rts_p = jnp.pad(edge_weight_rates, pad_i)

    agg_u0, agg_a0, den_u, den_a = _agg_kernel_l0(
        x_artwork, x_user, sd_rev, ew_rev_p, sd_rts, ew_rts_p, z2, z1)

    den_u2 = den_u.reshape(N_NODE, 1)
    den_a2 = den_a.reshape(N_NODE, 1)
    bu0 = b_user_l0.reshape(1, H)
    ba0 = b_art_l0.reshape(1, H)
    hu, ha = _l0_call(
        x_user, x_artwork, agg_u0, agg_a0, den_u2, den_a2,
        W_self_user_l0, W_neigh_user_l0, bu0,
        W_self_art_l0, W_neigh_art_l0, ba0)

    agg_u1, agg_a1 = _agg_kernel_l1(
        ha, hu, sd_rev, ew_rev_p, sd_rts, ew_rts_p, z2, z1)

    su, sa = _l1_call(
        hu, ha, agg_u1, agg_a1, den_u2, den_a2,
        W_self_user_l1, W_neigh_user_l1, b_user_l1.reshape(1, H),
        W_self_art_l1, W_neigh_art_l1, b_art_l1.reshape(1, H),
        lin_user_W, lin_user_b.reshape(1, H // 2),
        lin_item_W, lin_item_b.reshape(1, H // 2),
        out_W[:H // 2], out_W[H // 2:], out_b.reshape(1, 1))

    out = _score_kernel(su.reshape(N_NODE), sa.reshape(N_NODE),
                        src_rts, dst_rts)
    return out.reshape(E, 1)


# speedup vs baseline: 1.5093x; 1.0000x over previous
import jax
import jax.numpy as jnp
from jax import lax
from jax.experimental import pallas as pl
from jax.experimental.pallas import tpu as pltpu
from jax.experimental.pallas import tpu_sc as plsc

N_NODE = 10000
E = 320000
D = 128
H = 128

NC = 2
NS = 16
NW = NC * NS

CHUNK = 120
NCH = 168
E_PAD = NCH * NS * CHUNK
NCHT = NCH * NS
RPT = 624
RPT_LAST = N_NODE - RPT * (NS - 1)


def _tiled_rowcopy(src, dst, s):
    start = pl.multiple_of(RPT * s, 8)

    @pl.when(s < NS - 1)
    def _():
        pltpu.sync_copy(src.at[pl.ds(start, RPT)], dst.at[pl.ds(start, RPT)])

    @pl.when(s == NS - 1)
    def _():
        pltpu.sync_copy(src.at[pl.ds(RPT * (NS - 1), RPT_LAST)],
                        dst.at[pl.ds(RPT * (NS - 1), RPT_LAST)])


def _agg_side(table, sdr, ewr, agg_out, den_out, z2, z1,
              agg_sh, den_sh, ibufs, dvec, s, with_den):
    _tiled_rowcopy(z2, agg_sh, s)
    if with_den:
        @pl.when(s < 10)
        def _():
            pltpu.sync_copy(z1.at[pl.ds(1000 * s, 1000)], dvec)
            pltpu.sync_copy(dvec, den_sh.at[pl.ds(1000 * s, 1000)])
    plsc.subcore_barrier()

    base0 = s * (NCH * CHUNK)
    cid0 = s * NCH

    def issue_idx(j, buf):
        sd_b, ew_b, rows_b, isem, gsem, ssem = buf
        base = pl.multiple_of(base0 + j * CHUNK, 8)
        pltpu.async_copy(sdr.at[cid0 + j], sd_b, isem)
        pltpu.async_copy(ewr.at[pl.ds(base, CHUNK)], ew_b, isem)

    def wait_idx(j, buf):
        sd_b, ew_b, rows_b, isem, gsem, ssem = buf
        base = pl.multiple_of(base0 + j * CHUNK, 8)
        pltpu.make_async_copy(sdr.at[cid0 + j], sd_b, isem).wait()
        pltpu.make_async_copy(ewr.at[pl.ds(base, CHUNK)], ew_b, isem).wait()

    def issue_gather(buf):
        sd_b, ew_b, rows_b, isem, gsem, ssem = buf
        pltpu.async_copy(table.at[sd_b.at[0]], rows_b, gsem)

    def wait_scatter(buf):
        sd_b, ew_b, rows_b, isem, gsem, ssem = buf
        pltpu.make_async_copy(rows_b, agg_sh.at[sd_b.at[1]], ssem).wait()
        if with_den:
            pltpu.make_async_copy(ew_b, den_sh.at[sd_b.at[1]], ssem).wait()

    def consume(buf):
        sd_b, ew_b, rows_b, isem, gsem, ssem = buf
        pltpu.make_async_copy(table.at[sd_b.at[0]], rows_b, gsem).wait()

        def scale(e, carry):
            w = plsc.load_gather(ew_b, [jnp.full((16,), e, jnp.int32)])
            for k in range(H // 16):
                sl = pl.ds(16 * k, 16)
                rows_b[e, sl] = rows_b[e, sl] * w
            return carry
        lax.fori_loop(0, CHUNK, scale, 0, unroll=8)

        pltpu.async_copy(rows_b, agg_sh.at[sd_b.at[1]], ssem, add=True)
        if with_den:
            pltpu.async_copy(ew_b, den_sh.at[sd_b.at[1]], ssem, add=True)

    bufs = ibufs
    issue_idx(0, bufs[0])
    wait_idx(0, bufs[0])
    issue_gather(bufs[0])
    issue_idx(1, bufs[1])

    def step(j, u):
        bj = bufs[u]
        bn = bufs[(u + 1) % 3]
        bp = bufs[(u + 2) % 3]

        @pl.when(j < NCH - 1)
        def _():
            wait_idx(j + 1, bn)
            issue_gather(bn)
        consume(bj)

        @pl.when(j > 0)
        def _():
            wait_scatter(bp)

        @pl.when(j < NCH - 2)
        def _():
            issue_idx(j + 2, bp)

    def triple(jt, carry):
        for u in range(3):
            step(3 * jt + u, u)
        return carry
    lax.fori_loop(0, NCH // 3, triple, 0)
    wait_scatter(bufs[(NCH - 1) % 3])

    plsc.subcore_barrier()
    _tiled_rowcopy(agg_sh, agg_out, s)
    if with_den:
        @pl.when(s < 10)
        def _():
            pltpu.sync_copy(den_sh.at[pl.ds(1000 * s, 1000)], dvec)
            pltpu.sync_copy(dvec, den_out.at[pl.ds(1000 * s, 1000)])


def _make_agg_kernel(with_den):
    mesh = plsc.VectorSubcoreMesh(core_axis_name="c", subcore_axis_name="s")

    outs = [jax.ShapeDtypeStruct((N_NODE, H), jnp.float32),
            jax.ShapeDtypeStruct((N_NODE, H), jnp.float32)]
    if with_den:
        outs += [jax.ShapeDtypeStruct((N_NODE,), jnp.float32),
                 jax.ShapeDtypeStruct((N_NODE,), jnp.float32)]

    buf_types = [
        pltpu.VMEM((2, CHUNK), jnp.int32),
        pltpu.VMEM((CHUNK,), jnp.float32),
        pltpu.VMEM((CHUNK, H), jnp.float32),
        pltpu.SemaphoreType.DMA,
        pltpu.SemaphoreType.DMA,
        pltpu.SemaphoreType.DMA,
    ]
    scratch = (
        [pltpu.VMEM_SHARED((N_NODE, H), jnp.float32),
         pltpu.VMEM_SHARED((N_NODE,), jnp.float32)]
        + buf_types * 3
        + [pltpu.VMEM((1000,), jnp.float32)]
    )

    def body(tab0, tab1, sd0, ew0, sd1, ew1, z2, z1, *rest):
        if with_den:
            agg_u, agg_a, den_u, den_a = rest[:4]
            scr = rest[4:]
        else:
            agg_u, agg_a = rest[:2]
            den_u = den_a = None
            scr = rest[2:]
        agg_sh, den_sh = scr[0], scr[1]
        ibufs = tuple(tuple(scr[2 + 6 * i:8 + 6 * i]) for i in range(3))
        dvec = scr[20]
        c = lax.axis_index("c")
        s = lax.axis_index("s")

        @pl.when(c == 0)
        def _():
            _agg_side(tab0, sd0, ew0, agg_u, den_u, z2, z1,
                      agg_sh, den_sh, ibufs, dvec, s, with_den)

        @pl.when(c == 1)
        def _():
            _agg_side(tab1, sd1, ew1, agg_a, den_a, z2, z1,
                      agg_sh, den_sh, ibufs, dvec, s, with_den)

    return pl.kernel(body, out_type=tuple(outs), mesh=mesh,
                     scratch_types=scratch,
                     compiler_params=pltpu.CompilerParams(
                         needs_layout_passes=False),
                     name="sc_edge_agg_den" if with_den else "sc_edge_agg")


_agg_kernel_l0 = _make_agg_kernel(True)
_agg_kernel_l1 = _make_agg_kernel(False)


EPW = E // NW


def _score_body(su_hbm, sa_hbm, src_hbm, dst_hbm, out_hbm,
                su_v, sa_v, src_v, dst_v, out_v):
    c = lax.axis_index("c")
    s = lax.axis_index("s")
    w = s * NC + c
    pltpu.sync_copy(su_hbm, su_v)
    pltpu.sync_copy(sa_hbm, sa_v)
    base = w * EPW
    pltpu.sync_copy(src_hbm.at[pl.ds(base, EPW)], src_v)
    pltpu.sync_copy(dst_hbm.at[pl.ds(base, EPW)], dst_v)

    def it(i, carry):
        sl = pl.ds(i * 16, 16)
        vu = plsc.load_gather(su_v, [src_v[sl]])
        va = plsc.load_gather(sa_v, [dst_v[sl]])
        out_v[sl] = vu + va
        return carry
    lax.fori_loop(0, EPW // 16, it, 0)
    pltpu.sync_copy(out_v, out_hbm.at[pl.ds(base, EPW)])


_score_kernel = pl.kernel(
    _score_body,
    out_type=jax.ShapeDtypeStruct((E,), jnp.float32),
    mesh=plsc.VectorSubcoreMesh(core_axis_name="c", subcore_axis_name="s"),
    scratch_types=[
        pltpu.VMEM((N_NODE,), jnp.float32),
        pltpu.VMEM((N_NODE,), jnp.float32),
        pltpu.VMEM((EPW,), jnp.int32),
        pltpu.VMEM((EPW,), jnp.int32),
        pltpu.VMEM((EPW,), jnp.float32),
    ],
    compiler_params=pltpu.CompilerParams(needs_layout_passes=False),
    name="sc_edge_score",
)


BR = 1000
GRID = N_NODE // BR


def _l0_body(xu, xa, aggu, agga, denu, dena,
             Wsu, Wnu, bu, Wsa, Wna, ba, hu_ref, ha_ref):
    du = jnp.maximum(denu[...], 1e-6)
    da = jnp.maximum(dena[...], 1e-6)
    au = aggu[...] / du
    aa = agga[...] / da
    hu_ref[...] = jnp.maximum(xu[...] @ Wsu[...] + au @ Wnu[...] + bu[...], 0.0)
    ha_ref[...] = jnp.maximum(xa[...] @ Wsa[...] + aa @ Wna[...] + ba[...], 0.0)


def _l1_body(hu, ha, aggu, agga, denu, dena,
             Wsu, Wnu, bu, Wsa, Wna, ba,
             luW, lub, liW, lib, oWu, oWi, ob, su_ref, sa_ref):
    du = jnp.maximum(denu[...], 1e-6)
    da = jnp.maximum(dena[...], 1e-6)
    zu = hu[...] @ Wsu[...] + (aggu[...] / du) @ Wnu[...] + bu[...]
    za = ha[...] @ Wsa[...] + (agga[...] / da) @ Wna[...] + ba[...]
    uf = zu @ luW[...] + lub[...]
    itf = za @ liW[...] + lib[...]
    su_ref[...] = uf @ oWu[...] + ob[...]
    sa_ref[...] = itf @ oWi[...]


def _row_spec(ncol):
    return pl.BlockSpec((BR, ncol), lambda i: (i, 0))


def _full_spec(shape):
    nd = len(shape)
    return pl.BlockSpec(shape, lambda i: (0,) * nd)


_l0_call = pl.pallas_call(
    _l0_body,
    grid=(GRID,),
    in_specs=[_row_spec(H), _row_spec(H), _row_spec(H), _row_spec(H),
              _row_spec(1), _row_spec(1),
              _full_spec((H, H)), _full_spec((H, H)), _full_spec((1, H)),
              _full_spec((H, H)), _full_spec((H, H)), _full_spec((1, H))],
    out_specs=[_row_spec(H), _row_spec(H)],
    out_shape=[jax.ShapeDtypeStruct((N_NODE, H), jnp.float32),
               jax.ShapeDtypeStruct((N_NODE, H), jnp.float32)],
)

_l1_call = pl.pallas_call(
    _l1_body,
    grid=(GRID,),
    in_specs=[_row_spec(H), _row_spec(H), _row_spec(H), _row_spec(H),
              _row_spec(1), _row_spec(1),
              _full_spec((H, H)), _full_spec((H, H)), _full_spec((1, H)),
              _full_spec((H, H)), _full_spec((H, H)), _full_spec((1, H)),
              _full_spec((H, H // 2)), _full_spec((1, H // 2)),
              _full_spec((H, H // 2)), _full_spec((1, H // 2)),
              _full_spec((H // 2, 1)), _full_spec((H // 2, 1)),
              _full_spec((1, 1))],
    out_specs=[_row_spec(1), _row_spec(1)],
    out_shape=[jax.ShapeDtypeStruct((N_NODE, 1), jnp.float32),
               jax.ShapeDtypeStruct((N_NODE, 1), jnp.float32)],
)


def kernel(x_user, x_artwork, edge_index_rates, edge_index_rev,
           edge_weight_rates, edge_weight_rev,
           W_self_user_l0, W_neigh_user_l0, b_user_l0,
           W_self_art_l0, W_neigh_art_l0, b_art_l0,
           W_self_user_l1, W_neigh_user_l1, b_user_l1,
           W_self_art_l1, W_neigh_art_l1, b_art_l1,
           lin_user_W, lin_user_b, lin_item_W, lin_item_b, out_W, out_b):
    src_rev = edge_index_rev[0]
    dst_rev = edge_index_rev[1]
    src_rts = edge_index_rates[0]
    dst_rts = edge_index_rates[1]
    z2 = jnp.zeros((N_NODE, H), jnp.float32)
    z1 = jnp.zeros((N_NODE,), jnp.float32)

    pad_i = [(0, E_PAD - E)]

    def pack_sd(src, dst):
        return jnp.stack([jnp.pad(src, pad_i).reshape(NCHT, CHUNK),
                          jnp.pad(dst, pad_i).reshape(NCHT, CHUNK)], axis=1)

    sd_rev = pack_sd(src_rev, dst_rev)
    ew_rev_p = jnp.pad(edge_weight_rev, pad_i)
    sd_rts = pack_sd(src_rts, dst_rts)
    ew_rts_p = jnp.pad(edge_weight_rates, pad_i)

    agg_u0, agg_a0, den_u, den_a = _agg_kernel_l0(
        x_artwork, x_user, sd_rev, ew_rev_p, sd_rts, ew_rts_p, z2, z1)

    den_u2 = den_u.reshape(N_NODE, 1)
    den_a2 = den_a.reshape(N_NODE, 1)
    bu0 = b_user_l0.reshape(1, H)
    ba0 = b_art_l0.reshape(1, H)
    hu, ha = _l0_call(
        x_user, x_artwork, agg_u0, agg_a0, den_u2, den_a2,
        W_self_user_l0, W_neigh_user_l0, bu0,
        W_self_art_l0, W_neigh_art_l0, ba0)

    agg_u1, agg_a1 = _agg_kernel_l1(
        ha, hu, sd_rev, ew_rev_p, sd_rts, ew_rts_p, z2, z1)

    su, sa = _l1_call(
        hu, ha, agg_u1, agg_a1, den_u2, den_a2,
        W_self_user_l1, W_neigh_user_l1, b_user_l1.reshape(1, H),
        W_self_art_l1, W_neigh_art_l1, b_art_l1.reshape(1, H),
        lin_user_W, lin_user_b.reshape(1, H // 2),
        lin_item_W, lin_item_b.reshape(1, H // 2),
        out_W[:H // 2], out_W[H // 2:], out_b.reshape(1, 1))

    out = _score_kernel(su.reshape(N_NODE), sa.reshape(N_NODE),
                        src_rts, dst_rts)
    return out.reshape(E, 1)
